# baseline (speedup 1.0000x reference)
import functools

import jax
import jax.numpy as jnp
from jax import lax
from jax.experimental import pallas as pl
from jax.experimental.pallas import tpu as pltpu
from jax.experimental.pallas import tpu_sc as plsc

P = 32768
MT = 8192
MS = 256
NC = 2
NS = 16
L = 16
E = MT // NS
QC = P // NS
QCHUNK = 1024
INVALID_TOKEN = -1


BPL = E // L


def _body(qv, qt, qs, qp, params,
          o_tok, o_seq, o_pos, o_val,
          nq_tok, nq_seq, nq_pos, nq_val, o_cnt,
          tok_v, seq_v, pos_v, val_v, rank_v, dest_v,
          hist2_v, prefix2_v, hist_v, allhist_v, cursor_v, counts_v, param_v,
          qtmp_i, qtmp_f, const_i, const_f, shared_hist, dma_sem):
    c = lax.axis_index("c")
    sid = lax.axis_index("s")
    pltpu.sync_copy(params, param_v)
    pvec = param_v[pl.ds(0, L)]
    num = pvec[0]
    max_seq = pvec[1]

    @pl.when(c == 0)
    def _sort_side():
        base = pl.multiple_of(sid * E, 8)
        pltpu.sync_copy(qt.at[pl.ds(base, E)], tok_v)
        pltpu.sync_copy(qs.at[pl.ds(base, E)], seq_v)
        pltpu.sync_copy(qp.at[pl.ds(base, E)], pos_v)
        pltpu.sync_copy(qv.at[pl.ds(base, E)], val_v)

        zero16 = jnp.zeros((L,), jnp.int32)
        iota16 = lax.iota(jnp.int32, L)

        @pl.loop(0, L * MS // L)
        def _zero(i):
            hist2_v[pl.ds(i * L, L)] = zero16

        @pl.loop(0, BPL)
        def _count(t):
            eidx = iota16 * BPL + t
            s = plsc.load_gather(seq_v, [eidx])
            hidx = iota16 * MS + s
            old = plsc.load_gather(hist2_v, [hidx])
            plsc.store_scatter(rank_v, [eidx], old)
            plsc.store_scatter(hist2_v, [hidx], old + 1)

        @pl.loop(0, MS // L)
        def _lane_prefix(g):
            acc = zero16
            for l in range(L):
                off = l * MS + g * L
                row = hist2_v[pl.ds(off, L)]
                prefix2_v[pl.ds(off, L)] = acc
                acc = acc + row
            hist_v[pl.ds(g * L, L)] = acc

        pltpu.sync_copy(hist_v, shared_hist.at[sid])
        plsc.subcore_barrier()
        pltpu.sync_copy(shared_hist, allhist_v)

        @pl.loop(0, MS // L, init_carry=jnp.int32(0))
        def _offsets(g, carry):
            sl = pl.ds(g * L, L)
            total = jnp.zeros((L,), jnp.int32)
            before = jnp.zeros((L,), jnp.int32)
            for w in range(NS):
                row = allhist_v[w, sl]
                total = total + row
                before = before + row * (jnp.int32(w) < sid).astype(jnp.int32)
            inc = plsc.cumsum(total)
            excl = inc - total + carry
            cursor_v[sl] = excl + before
            bidx = iota16 + g * L
            cnt = jnp.minimum(jnp.maximum(num - excl, 0), total)
            cnt = jnp.where(bidx < max_seq, cnt, zero16)
            counts_v[sl] = cnt
            return carry + jnp.sum(total)

        @pl.when(sid == 0)
        def _write_counts():
            pltpu.sync_copy(counts_v, o_cnt)

        for t in range(E // L):
            sl = pl.ds(t * L, L)
            sv = seq_v[sl]
            cur = plsc.load_gather(cursor_v, [sv])
            pre = plsc.load_gather(prefix2_v, [jnp.int32((t // 2) * MS) + sv])
            dest_v[t // 8, pl.ds((t % 8) * L, L)] = cur + pre + rank_v[sl]

        descs = []
        for src, dst in ((tok_v, o_tok), (seq_v, o_seq),
                         (pos_v, o_pos), (val_v, o_val)):
            for h in range(E // 128):
                descs.append(pltpu.async_copy(
                    src.at[pl.ds(h * 128, 128)],
                    dst.at[dest_v.at[h]], dma_sem))
        for d in descs:
            d.wait()

    @pl.when(c == 1)
    def _roll_side():
        neg16 = jnp.full((L,), INVALID_TOKEN, jnp.int32)
        zf16 = jnp.zeros((L,), jnp.float32)

        @pl.loop(0, QCHUNK // L)
        def _fill(i):
            const_i[pl.ds(i * L, L)] = neg16
            const_f[pl.ds(i * L, L)] = zf16

        cutoff = P - num
        for ck in range(QC // QCHUNK):
            b0 = sid * QC + ck * QCHUNK
            dst_off = pl.multiple_of(b0, 8)
            src_off = pl.multiple_of(b0 + num, 8)

            @pl.when(b0 < cutoff)
            def _copy():
                for src, dst, tmp in ((qt, nq_tok, qtmp_i),
                                      (qs, nq_seq, qtmp_i),
                                      (qp, nq_pos, qtmp_i),
                                      (qv, nq_val, qtmp_f)):
                    pltpu.sync_copy(src.at[pl.ds(src_off, QCHUNK)], tmp)
                    pltpu.sync_copy(tmp, dst.at[pl.ds(dst_off, QCHUNK)])

            @pl.when(b0 >= cutoff)
            def _mask():
                pltpu.sync_copy(const_i, nq_tok.at[pl.ds(dst_off, QCHUNK)])
                pltpu.sync_copy(const_i, nq_seq.at[pl.ds(dst_off, QCHUNK)])
                pltpu.sync_copy(const_i, nq_pos.at[pl.ds(dst_off, QCHUNK)])
                pltpu.sync_copy(const_f, nq_val.at[pl.ds(dst_off, QCHUNK)])


@jax.jit
def _packed(queued_values, queued_tokens, queued_seq_ids, queued_pos_ids,
            params):
    mesh = plsc.VectorSubcoreMesh(core_axis_name="c", subcore_axis_name="s",
                                  num_cores=NC, num_subcores=NS)
    i32 = jnp.int32
    f32 = jnp.float32
    run = pl.kernel(
        _body,
        out_type=(
            jax.ShapeDtypeStruct((MT,), i32),
            jax.ShapeDtypeStruct((MT,), i32),
            jax.ShapeDtypeStruct((MT,), i32),
            jax.ShapeDtypeStruct((MT,), f32),
            jax.ShapeDtypeStruct((P,), i32),
            jax.ShapeDtypeStruct((P,), i32),
            jax.ShapeDtypeStruct((P,), i32),
            jax.ShapeDtypeStruct((P,), f32),
            jax.ShapeDtypeStruct((MS,), i32),
        ),
        mesh=mesh,
        compiler_params=pltpu.CompilerParams(needs_layout_passes=False),
        scratch_types=[
            pltpu.VMEM((E,), i32),
            pltpu.VMEM((E,), i32),
            pltpu.VMEM((E,), i32),
            pltpu.VMEM((E,), f32),
            pltpu.VMEM((E,), i32),
            pltpu.VMEM((E // 128, 128), i32),
            pltpu.VMEM((L * MS,), i32),
            pltpu.VMEM((L * MS,), i32),
            pltpu.VMEM((MS,), i32),
            pltpu.VMEM((NS, MS), i32),
            pltpu.VMEM((MS,), i32),
            pltpu.VMEM((MS,), i32),
            pltpu.VMEM((16,), i32),
            pltpu.VMEM((QCHUNK,), i32),
            pltpu.VMEM((QCHUNK,), f32),
            pltpu.VMEM((QCHUNK,), i32),
            pltpu.VMEM((QCHUNK,), f32),
            pltpu.VMEM_SHARED((NS, MS), i32),
            pltpu.SemaphoreType.DMA,
        ],
    )
    return run(queued_values, queued_tokens, queued_seq_ids, queued_pos_ids,
               params)


def kernel(queued_values, queued_tokens, queued_seq_ids, queued_pos_ids,
           num_queued_tokens, max_tokens, max_sequences):
    nqt = jnp.asarray(num_queued_tokens, jnp.int32)
    mt = jnp.asarray(max_tokens, jnp.int32)
    ms = jnp.asarray(max_sequences, jnp.int32)
    num = jnp.minimum(nqt, mt)
    pad = jnp.zeros((14,), jnp.int32)
    params = jnp.concatenate([num[None], ms[None], pad])
    (tokens, seq_ids, pos_ids, vals,
     nq_tok, nq_seq, nq_pos, nq_val, counts) = _packed(
        queued_values, queued_tokens, queued_seq_ids, queued_pos_ids, params)
    return (tokens, seq_ids, pos_ids, vals,
            nq_tok, nq_seq, nq_pos, nq_val, counts, num, nqt - num)


# speedup vs baseline: 1.0018x; 1.0018x over previous
import functools

import jax
import jax.numpy as jnp
from jax import lax
from jax.experimental import pallas as pl
from jax.experimental.pallas import tpu as pltpu
from jax.experimental.pallas import tpu_sc as plsc

P = 32768
MT = 8192
MS = 256
NC = 2
NS = 16
L = 16
E = MT // NS
QC = P // NS
QCHUNK = 1024
INVALID_TOKEN = -1


BPL = E // L


def _body(qv, qt, qs, qp, params,
          o_tok, o_seq, o_pos, o_val,
          nq_tok, nq_seq, nq_pos, nq_val, o_cnt,
          tok_v, seq_v, pos_v, val_v, rank_v, dest_v,
          hist2_v, prefix2_v, hist_v, allhist_v, cursor_v, counts_v, param_v,
          qtmp_i, qtmp_f, const_i, const_f, shared_hist, dma_sem):
    c = lax.axis_index("c")
    sid = lax.axis_index("s")
    pltpu.sync_copy(params, param_v)
    pvec = param_v[pl.ds(0, L)]
    num = pvec[0]
    max_seq = pvec[1]

    @pl.when(c == 0)
    def _sort_side():
        base = pl.multiple_of(sid * E, 8)
        with jax.named_scope("ph_stage_in"):
            pltpu.sync_copy(qt.at[pl.ds(base, E)], tok_v)
            pltpu.sync_copy(qs.at[pl.ds(base, E)], seq_v)
            pltpu.sync_copy(qp.at[pl.ds(base, E)], pos_v)
            pltpu.sync_copy(qv.at[pl.ds(base, E)], val_v)

        zero16 = jnp.zeros((L,), jnp.int32)
        iota16 = lax.iota(jnp.int32, L)

        with jax.named_scope("ph_zero"):
            @pl.loop(0, L * MS // L)
            def _zero(i):
                hist2_v[pl.ds(i * L, L)] = zero16

        with jax.named_scope("ph_hist"):
            @pl.loop(0, BPL)
            def _count(t):
                eidx = iota16 * BPL + t
                s = plsc.load_gather(seq_v, [eidx])
                hidx = iota16 * MS + s
                old = plsc.load_gather(hist2_v, [hidx])
                plsc.store_scatter(rank_v, [eidx], old)
                plsc.store_scatter(hist2_v, [hidx], old + 1)

        with jax.named_scope("ph_lane_prefix"):
            @pl.loop(0, MS // L)
            def _lane_prefix(g):
                acc = zero16
                for l in range(L):
                    off = l * MS + g * L
                    row = hist2_v[pl.ds(off, L)]
                    prefix2_v[pl.ds(off, L)] = acc
                    acc = acc + row
                hist_v[pl.ds(g * L, L)] = acc

        with jax.named_scope("ph_exchange"):
            pltpu.sync_copy(hist_v, shared_hist.at[sid])
            plsc.subcore_barrier()
            pltpu.sync_copy(shared_hist, allhist_v)

        with jax.named_scope("ph_offsets"):
            @pl.loop(0, MS // L, init_carry=jnp.int32(0))
            def _offsets(g, carry):
                sl = pl.ds(g * L, L)
                total = jnp.zeros((L,), jnp.int32)
                before = jnp.zeros((L,), jnp.int32)
                for w in range(NS):
                    row = allhist_v[w, sl]
                    total = total + row
                    before = before + row * (jnp.int32(w) < sid).astype(jnp.int32)
                inc = plsc.cumsum(total)
                excl = inc - total + carry
                cursor_v[sl] = excl + before
                bidx = iota16 + g * L
                cnt = jnp.minimum(jnp.maximum(num - excl, 0), total)
                cnt = jnp.where(bidx < max_seq, cnt, zero16)
                counts_v[sl] = cnt
                return carry + jnp.sum(total)

            @pl.when(sid == 0)
            def _write_counts():
                pltpu.sync_copy(counts_v, o_cnt)

        with jax.named_scope("ph_dest"):
            for t in range(E // L):
                sl = pl.ds(t * L, L)
                sv = seq_v[sl]
                cur = plsc.load_gather(cursor_v, [sv])
                pre = plsc.load_gather(prefix2_v,
                                       [jnp.int32((t // 2) * MS) + sv])
                dest_v[t // 8, pl.ds((t % 8) * L, L)] = cur + pre + rank_v[sl]

        with jax.named_scope("ph_scatter"):
            descs = []
            for src, dst in ((tok_v, o_tok), (seq_v, o_seq),
                             (pos_v, o_pos), (val_v, o_val)):
                for h in range(E // 128):
                    descs.append(pltpu.async_copy(
                        src.at[pl.ds(h * 128, 128)],
                        dst.at[dest_v.at[h]], dma_sem))
            for d in descs:
                d.wait()

    @pl.when(c == 1)
    def _roll_side():
        neg16 = jnp.full((L,), INVALID_TOKEN, jnp.int32)
        zf16 = jnp.zeros((L,), jnp.float32)

        @pl.loop(0, QCHUNK // L)
        def _fill(i):
            const_i[pl.ds(i * L, L)] = neg16
            const_f[pl.ds(i * L, L)] = zf16

        cutoff = P - num
        for ck in range(QC // QCHUNK):
            b0 = sid * QC + ck * QCHUNK
            dst_off = pl.multiple_of(b0, 8)
            src_off = pl.multiple_of(b0 + num, 8)

            @pl.when(b0 < cutoff)
            def _copy():
                for src, dst, tmp in ((qt, nq_tok, qtmp_i),
                                      (qs, nq_seq, qtmp_i),
                                      (qp, nq_pos, qtmp_i),
                                      (qv, nq_val, qtmp_f)):
                    pltpu.sync_copy(src.at[pl.ds(src_off, QCHUNK)], tmp)
                    pltpu.sync_copy(tmp, dst.at[pl.ds(dst_off, QCHUNK)])

            @pl.when(b0 >= cutoff)
            def _mask():
                pltpu.sync_copy(const_i, nq_tok.at[pl.ds(dst_off, QCHUNK)])
                pltpu.sync_copy(const_i, nq_seq.at[pl.ds(dst_off, QCHUNK)])
                pltpu.sync_copy(const_i, nq_pos.at[pl.ds(dst_off, QCHUNK)])
                pltpu.sync_copy(const_f, nq_val.at[pl.ds(dst_off, QCHUNK)])


@jax.jit
def _packed(queued_values, queued_tokens, queued_seq_ids, queued_pos_ids,
            params):
    mesh = plsc.VectorSubcoreMesh(core_axis_name="c", subcore_axis_name="s",
                                  num_cores=NC, num_subcores=NS)
    i32 = jnp.int32
    f32 = jnp.float32
    run = pl.kernel(
        _body,
        out_type=(
            jax.ShapeDtypeStruct((MT,), i32),
            jax.ShapeDtypeStruct((MT,), i32),
            jax.ShapeDtypeStruct((MT,), i32),
            jax.ShapeDtypeStruct((MT,), f32),
            jax.ShapeDtypeStruct((P,), i32),
            jax.ShapeDtypeStruct((P,), i32),
            jax.ShapeDtypeStruct((P,), i32),
            jax.ShapeDtypeStruct((P,), f32),
            jax.ShapeDtypeStruct((MS,), i32),
        ),
        mesh=mesh,
        compiler_params=pltpu.CompilerParams(needs_layout_passes=False),
        scratch_types=[
            pltpu.VMEM((E,), i32),
            pltpu.VMEM((E,), i32),
            pltpu.VMEM((E,), i32),
            pltpu.VMEM((E,), f32),
            pltpu.VMEM((E,), i32),
            pltpu.VMEM((E // 128, 128), i32),
            pltpu.VMEM((L * MS,), i32),
            pltpu.VMEM((L * MS,), i32),
            pltpu.VMEM((MS,), i32),
            pltpu.VMEM((NS, MS), i32),
            pltpu.VMEM((MS,), i32),
            pltpu.VMEM((MS,), i32),
            pltpu.VMEM((16,), i32),
            pltpu.VMEM((QCHUNK,), i32),
            pltpu.VMEM((QCHUNK,), f32),
            pltpu.VMEM((QCHUNK,), i32),
            pltpu.VMEM((QCHUNK,), f32),
            pltpu.VMEM_SHARED((NS, MS), i32),
            pltpu.SemaphoreType.DMA,
        ],
    )
    return run(queued_values, queued_tokens, queued_seq_ids, queued_pos_ids,
               params)


def kernel(queued_values, queued_tokens, queued_seq_ids, queued_pos_ids,
           num_queued_tokens, max_tokens, max_sequences):
    nqt = jnp.asarray(num_queued_tokens, jnp.int32)
    mt = jnp.asarray(max_tokens, jnp.int32)
    ms = jnp.asarray(max_sequences, jnp.int32)
    num = jnp.minimum(nqt, mt)
    pad = jnp.zeros((14,), jnp.int32)
    params = jnp.concatenate([num[None], ms[None], pad])
    (tokens, seq_ids, pos_ids, vals,
     nq_tok, nq_seq, nq_pos, nq_val, counts) = _packed(
        queued_values, queued_tokens, queued_seq_ids, queued_pos_ids, params)
    return (tokens, seq_ids, pos_ids, vals,
            nq_tok, nq_seq, nq_pos, nq_val, counts, num, nqt - num)
